# baseline (speedup 1.0000x reference)
import functools

import jax
import jax.numpy as jnp
from jax import lax
from jax.experimental import pallas as pl
from jax.experimental.pallas import tpu as pltpu
from jax.experimental.pallas import tpu_sc as plsc

_N = 10000
_E = 320000
_D = 128
_H = 128
_OUT = 32
_G = 64

_NCORE = 2
_NSUB = 16
_NW = _NCORE * _NSUB
_NPAD = 10240
_RPS = _NPAD // _NSUB
_CHUNK = 128
_CPW = 80
_EPAD = _NW * _CPW * _CHUNK
_NB = 2
_NBLK = _CPW // _NB

_mesh = plsc.VectorSubcoreMesh(core_axis_name="c", subcore_axis_name="s")


@functools.partial(
    pl.kernel,
    out_type=jax.ShapeDtypeStruct((_NCORE, _NPAD), jnp.float32),
    mesh=_mesh,
    scratch_types=[
        pltpu.VMEM((_CPW, _CHUNK), jnp.int32),
        pltpu.VMEM((_CHUNK,), jnp.float32),
        pltpu.VMEM_SHARED((_NPAD,), jnp.float32),
    ],
)
def _deg_pass(col2d_hbm, zero1_hbm, out_hbm, cidx, ones_v, acc):
    c = lax.axis_index("c")
    s = lax.axis_index("s")
    w = s * _NCORE + c
    for i in range(_CHUNK // 16):
        ones_v[pl.ds(i * 16, 16)] = jnp.ones((16,), jnp.float32)
    pltpu.sync_copy(col2d_hbm.at[pl.ds(w * _CPW, _CPW)], cidx)
    pltpu.sync_copy(zero1_hbm, acc.at[pl.ds(s * _RPS, _RPS)])
    plsc.subcore_barrier()

    def body(k, carry):
        pltpu.sync_copy(ones_v, acc.at[cidx.at[k]], add=True)
        return carry

    lax.fori_loop(0, _CPW, body, 0)
    plsc.subcore_barrier()
    pltpu.sync_copy(acc.at[pl.ds(s * _RPS, _RPS)],
                    out_hbm.at[c, pl.ds(s * _RPS, _RPS)])


@functools.partial(
    pl.kernel,
    out_type=jax.ShapeDtypeStruct((_NCORE, _NPAD, _H), jnp.float32),
    mesh=_mesh,
    scratch_types=[
        pltpu.VMEM((_CPW, _CHUNK), jnp.int32),
        pltpu.VMEM((_CPW, _CHUNK), jnp.int32),
        pltpu.VMEM((_CHUNK, _H), jnp.float32),
        pltpu.VMEM_SHARED((_NPAD, _H), jnp.float32),
        pltpu.SemaphoreType.DMA,
    ],
)
def _edge_pass(row2d_hbm, col2d_hbm, y_hbm, zero2_hbm, out_hbm,
               ridx, cidx, rows, acc, sem):
    c = lax.axis_index("c")
    s = lax.axis_index("s")
    w = s * _NCORE + c
    pltpu.sync_copy(row2d_hbm.at[pl.ds(w * _CPW, _CPW)], ridx)
    pltpu.sync_copy(col2d_hbm.at[pl.ds(w * _CPW, _CPW)], cidx)
    pltpu.sync_copy(zero2_hbm, acc.at[pl.ds(s * _RPS, _RPS)])
    plsc.subcore_barrier()

    def body(k, carry):
        pltpu.async_copy(y_hbm.at[ridx.at[k]], rows, sem).wait()
        pltpu.sync_copy(rows, acc.at[cidx.at[k]], add=True)
        return carry

    lax.fori_loop(0, _CPW, body, 0)
    plsc.subcore_barrier()
    pltpu.sync_copy(acc.at[pl.ds(s * _RPS, _RPS)],
                    out_hbm.at[c, pl.ds(s * _RPS, _RPS)])


_R = 1000
_GRID = _N // _R


def _pre_body(degp, nf, w0, b0, wc0, xo, y0o):
    d = degp[...]
    dinv = lax.rsqrt(d[0] + d[1] + 1.0)
    x = lax.dot_general(nf[...], w0[...], (((1,), (1,)), ((), ())),
                        preferred_element_type=jnp.float32) + b0[...]
    xo[...] = x
    h0 = lax.dot_general(x, wc0[...], (((1,), (1,)), ((), ())),
                         preferred_element_type=jnp.float32)
    y0o[...] = dinv * h0


def _mid_body(degp, x, y0, s0p, bc0, wc1, h0ro, y1o):
    d = degp[...]
    dinv = lax.rsqrt(d[0] + d[1] + 1.0)
    sp = s0p[...]
    t = dinv * (sp[0] + sp[1] + y0[...]) + bc0[...]
    h0r = jnp.maximum(t, 0.0)
    h0ro[...] = h0r
    emb = jnp.concatenate([x[...], h0r], axis=1)
    h1 = lax.dot_general(emb, wc1[...], (((1,), (1,)), ((), ())),
                         preferred_element_type=jnp.float32)
    y1o[...] = dinv * h1


def _fin_body(degp, x, h0r, y1, s1p, bc1, bt, wp1, bp1, wp2, bp2,
              out, pooled):
    i = pl.program_id(0)
    d = degp[...]
    dinv = lax.rsqrt(d[0] + d[1] + 1.0)
    sp = s1p[...]
    t = dinv * (sp[0] + sp[1] + y1[...]) + bc1[...]
    h1r = jnp.maximum(t, 0.0)
    emb = jnp.concatenate([x[...], h0r[...], h1r], axis=1)
    seg = lax.broadcasted_iota(jnp.int32, (_R, _G), 1)
    onehot = jnp.where(bt[...] == seg, 1.0, 0.0).astype(jnp.float32)
    part = lax.dot_general(onehot, emb, (((0,), (0,)), ((), ())),
                           preferred_element_type=jnp.float32)

    @pl.when(i == 0)
    def _():
        pooled[...] = part

    @pl.when(i > 0)
    def _():
        pooled[...] = pooled[...] + part

    @pl.when(i == _GRID - 1)
    def _():
        p = pooled[...]
        h = lax.dot_general(p, wp1[...], (((1,), (1,)), ((), ())),
                            preferred_element_type=jnp.float32) + bp1[...]
        h = jnp.where(h > 0, h, 0.1 * h)
        o = lax.dot_general(h, wp2[...], (((1,), (1,)), ((), ())),
                            preferred_element_type=jnp.float32) + bp2[...]
        m = jnp.max(o, axis=1, keepdims=True)
        lse = jnp.log(jnp.sum(jnp.exp(o - m), axis=1, keepdims=True)) + m
        out[...] = o - lse


def kernel(node_feature, edge_index, batch, W0, b0, Wc0, bc0, Wc1, bc1,
           Wp1, bp1, Wp2, bp2):
    f32 = jnp.float32
    i32 = jnp.int32
    pad = _EPAD - _E
    row2d = jnp.concatenate(
        [edge_index[0], jnp.zeros((pad,), i32)]).reshape(-1, _CHUNK)
    col2d = jnp.concatenate(
        [edge_index[1], jnp.full((pad,), _N, i32)]).reshape(-1, _CHUNK)
    zero1 = jnp.zeros((_RPS,), f32)
    zero2 = jnp.zeros((_RPS, _H), f32)

    deg_p = _deg_pass(col2d, zero1)
    degp3 = deg_p.reshape(_NCORE, _NPAD, 1)

    dspec = pl.BlockSpec((_NCORE, _R, 1), lambda i: (0, i, 0))
    rspec = pl.BlockSpec((_R, _H), lambda i: (i, 0))
    sspec = pl.BlockSpec((_NCORE, _R, _H), lambda i: (0, i, 0))

    x, y0 = pl.pallas_call(
        _pre_body,
        grid=(_GRID,),
        in_specs=[
            dspec,
            pl.BlockSpec((_R, _D), lambda i: (i, 0)),
            pl.BlockSpec((_H, _D), lambda i: (0, 0)),
            pl.BlockSpec((1, _H), lambda i: (0, 0)),
            pl.BlockSpec((_H, _H), lambda i: (0, 0)),
        ],
        out_specs=[rspec, rspec],
        out_shape=[jax.ShapeDtypeStruct((_N, _H), f32)] * 2,
    )(degp3, node_feature, W0, b0.reshape(1, _H), Wc0)

    s0_p = _edge_pass(row2d, col2d, y0, zero2)

    h0r, y1 = pl.pallas_call(
        _mid_body,
        grid=(_GRID,),
        in_specs=[
            dspec, rspec, rspec, sspec,
            pl.BlockSpec((1, _H), lambda i: (0, 0)),
            pl.BlockSpec((_H, 2 * _H), lambda i: (0, 0)),
        ],
        out_specs=[rspec, rspec],
        out_shape=[jax.ShapeDtypeStruct((_N, _H), f32)] * 2,
    )(degp3, x, y0, s0_p, bc0.reshape(1, _H), Wc1)

    s1_p = _edge_pass(row2d, col2d, y1, zero2)

    out = pl.pallas_call(
        _fin_body,
        grid=(_GRID,),
        in_specs=[
            dspec, rspec, rspec, rspec, sspec,
            pl.BlockSpec((1, _H), lambda i: (0, 0)),
            pl.BlockSpec((_R, 1), lambda i: (i, 0)),
            pl.BlockSpec((_H, 3 * _H), lambda i: (0, 0)),
            pl.BlockSpec((1, _H), lambda i: (0, 0)),
            pl.BlockSpec((_OUT, _H), lambda i: (0, 0)),
            pl.BlockSpec((1, _OUT), lambda i: (0, 0)),
        ],
        out_specs=pl.BlockSpec((_G, _OUT), lambda i: (0, 0)),
        out_shape=jax.ShapeDtypeStruct((_G, _OUT), f32),
        scratch_shapes=[pltpu.VMEM((_G, 3 * _H), f32)],
    )(degp3, x, h0r, y1, s1_p, bc1.reshape(1, _H),
      batch.reshape(_N, 1), Wp1, bp1.reshape(1, _H), Wp2,
      bp2.reshape(1, _OUT))
    return out


# speedup vs baseline: 1.1372x; 1.1372x over previous
import functools

import jax
import jax.numpy as jnp
from jax import lax
from jax.experimental import pallas as pl
from jax.experimental.pallas import tpu as pltpu
from jax.experimental.pallas import tpu_sc as plsc

_N = 10000
_E = 320000
_D = 128
_H = 128
_OUT = 32
_G = 64

_NCORE = 2
_NSUB = 16
_NW = _NCORE * _NSUB
_NPAD = 10240
_RPS = _NPAD // _NSUB
_CHUNK = 128
_CPW = 80
_EPAD = _NW * _CPW * _CHUNK
_HCP = 40

_mesh = plsc.VectorSubcoreMesh(core_axis_name="c", subcore_axis_name="s")


@functools.partial(
    pl.kernel,
    out_type=jax.ShapeDtypeStruct((_NCORE, _NPAD), jnp.float32),
    mesh=_mesh,
    scratch_types=[
        pltpu.VMEM((_CPW, _CHUNK), jnp.int32),
        pltpu.VMEM((_CHUNK,), jnp.float32),
        pltpu.VMEM_SHARED((_NPAD,), jnp.float32),
    ],
)
def _deg_pass(col2d_hbm, zero1_hbm, out_hbm, cidx, ones_v, acc):
    c = lax.axis_index("c")
    s = lax.axis_index("s")
    w = s * _NCORE + c
    for i in range(_CHUNK // 16):
        ones_v[pl.ds(i * 16, 16)] = jnp.ones((16,), jnp.float32)
    pltpu.sync_copy(col2d_hbm.at[pl.ds(w * _CPW, _CPW)], cidx)
    pltpu.sync_copy(zero1_hbm, acc.at[pl.ds(s * _RPS, _RPS)])
    plsc.subcore_barrier()

    def body(k, carry):
        pltpu.sync_copy(ones_v, acc.at[cidx.at[k]], add=True)
        return carry

    lax.fori_loop(0, _CPW, body, 0)
    plsc.subcore_barrier()
    pltpu.sync_copy(acc.at[pl.ds(s * _RPS, _RPS)],
                    out_hbm.at[c, pl.ds(s * _RPS, _RPS)])


@functools.partial(
    pl.kernel,
    out_type=jax.ShapeDtypeStruct((_NCORE, _NPAD, _H), jnp.float32),
    mesh=_mesh,
    scratch_types=[
        pltpu.VMEM((_HCP, _CHUNK), jnp.int32),
        pltpu.VMEM((_HCP, _CHUNK), jnp.int32),
        pltpu.VMEM((_CHUNK, _H), jnp.float32),
        pltpu.VMEM((_CHUNK, _H), jnp.float32),
        pltpu.VMEM_SHARED((_NPAD, _H), jnp.float32),
        pltpu.SemaphoreType.DMA,
        pltpu.SemaphoreType.DMA,
    ],
)
def _edge_pass(row2d_hbm, col2d_hbm, y_hbm, zero2_hbm, out_hbm,
               ridx, cidx, rows_a, rows_b, acc, sem_a, sem_b):
    c = lax.axis_index("c")
    s = lax.axis_index("s")
    w = s * _NCORE + c
    pltpu.sync_copy(zero2_hbm, acc.at[pl.ds(s * _RPS, _RPS)])
    plsc.subcore_barrier()

    for h in range(_CPW // _HCP):
        base = w * _CPW + h * _HCP
        pltpu.sync_copy(row2d_hbm.at[pl.ds(base, _HCP)], ridx)
        pltpu.sync_copy(col2d_hbm.at[pl.ds(base, _HCP)], cidx)
        pltpu.async_copy(y_hbm.at[ridx.at[0]], rows_a, sem_a)

        def body(k2, carry):
            kk = k2 * 2
            pltpu.async_copy(y_hbm.at[ridx.at[kk + 1]], rows_b, sem_b)
            pltpu.make_async_copy(y_hbm.at[ridx.at[kk]], rows_a,
                                  sem_a).wait()
            pltpu.sync_copy(rows_a, acc.at[cidx.at[kk]], add=True)
            pltpu.async_copy(y_hbm.at[ridx.at[kk + 2]], rows_a, sem_a)
            pltpu.make_async_copy(y_hbm.at[ridx.at[kk + 1]], rows_b,
                                  sem_b).wait()
            pltpu.sync_copy(rows_b, acc.at[cidx.at[kk + 1]], add=True)
            return carry

        lax.fori_loop(0, _HCP // 2 - 1, body, 0)
        pltpu.async_copy(y_hbm.at[ridx.at[_HCP - 1]], rows_b, sem_b)
        pltpu.make_async_copy(y_hbm.at[ridx.at[_HCP - 2]], rows_a,
                              sem_a).wait()
        pltpu.sync_copy(rows_a, acc.at[cidx.at[_HCP - 2]], add=True)
        pltpu.make_async_copy(y_hbm.at[ridx.at[_HCP - 1]], rows_b,
                              sem_b).wait()
        pltpu.sync_copy(rows_b, acc.at[cidx.at[_HCP - 1]], add=True)

    plsc.subcore_barrier()
    pltpu.sync_copy(acc.at[pl.ds(s * _RPS, _RPS)],
                    out_hbm.at[c, pl.ds(s * _RPS, _RPS)])


_R = 1000
_GRID = _N // _R


def _pre_body(degp, nf, w0, b0, wc0, xo, y0o):
    d = degp[...]
    dinv = lax.rsqrt(d[0] + d[1] + 1.0)
    x = lax.dot_general(nf[...], w0[...], (((1,), (1,)), ((), ())),
                        preferred_element_type=jnp.float32) + b0[...]
    xo[...] = x
    h0 = lax.dot_general(x, wc0[...], (((1,), (1,)), ((), ())),
                         preferred_element_type=jnp.float32)
    y0o[...] = dinv * h0


def _mid_body(degp, x, y0, s0p, bc0, wc1, h0ro, y1o):
    d = degp[...]
    dinv = lax.rsqrt(d[0] + d[1] + 1.0)
    sp = s0p[...]
    t = dinv * (sp[0] + sp[1] + y0[...]) + bc0[...]
    h0r = jnp.maximum(t, 0.0)
    h0ro[...] = h0r
    emb = jnp.concatenate([x[...], h0r], axis=1)
    h1 = lax.dot_general(emb, wc1[...], (((1,), (1,)), ((), ())),
                         preferred_element_type=jnp.float32)
    y1o[...] = dinv * h1


def _fin_body(degp, x, h0r, y1, s1p, bc1, bt, wp1, bp1, wp2, bp2,
              out, pooled):
    i = pl.program_id(0)
    d = degp[...]
    dinv = lax.rsqrt(d[0] + d[1] + 1.0)
    sp = s1p[...]
    t = dinv * (sp[0] + sp[1] + y1[...]) + bc1[...]
    h1r = jnp.maximum(t, 0.0)
    emb = jnp.concatenate([x[...], h0r[...], h1r], axis=1)
    seg = lax.broadcasted_iota(jnp.int32, (_R, _G), 1)
    onehot = jnp.where(bt[...] == seg, 1.0, 0.0).astype(jnp.float32)
    part = lax.dot_general(onehot, emb, (((0,), (0,)), ((), ())),
                           preferred_element_type=jnp.float32)

    @pl.when(i == 0)
    def _():
        pooled[...] = part

    @pl.when(i > 0)
    def _():
        pooled[...] = pooled[...] + part

    @pl.when(i == _GRID - 1)
    def _():
        p = pooled[...]
        h = lax.dot_general(p, wp1[...], (((1,), (1,)), ((), ())),
                            preferred_element_type=jnp.float32) + bp1[...]
        h = jnp.where(h > 0, h, 0.1 * h)
        o = lax.dot_general(h, wp2[...], (((1,), (1,)), ((), ())),
                            preferred_element_type=jnp.float32) + bp2[...]
        m = jnp.max(o, axis=1, keepdims=True)
        lse = jnp.log(jnp.sum(jnp.exp(o - m), axis=1, keepdims=True)) + m
        out[...] = o - lse


def kernel(node_feature, edge_index, batch, W0, b0, Wc0, bc0, Wc1, bc1,
           Wp1, bp1, Wp2, bp2):
    f32 = jnp.float32
    i32 = jnp.int32
    pad = _EPAD - _E
    row2d = jnp.concatenate(
        [edge_index[0], jnp.zeros((pad,), i32)]).reshape(-1, _CHUNK)
    dump = _N + (jnp.arange(pad, dtype=i32) % (_NPAD - _N))
    col2d = jnp.concatenate([edge_index[1], dump]).reshape(-1, _CHUNK)
    zero1 = jnp.zeros((_RPS,), f32)
    zero2 = jnp.zeros((_RPS, _H), f32)

    deg_p = _deg_pass(col2d, zero1)
    degp3 = deg_p.reshape(_NCORE, _NPAD, 1)

    dspec = pl.BlockSpec((_NCORE, _R, 1), lambda i: (0, i, 0))
    rspec = pl.BlockSpec((_R, _H), lambda i: (i, 0))
    sspec = pl.BlockSpec((_NCORE, _R, _H), lambda i: (0, i, 0))

    x, y0 = pl.pallas_call(
        _pre_body,
        grid=(_GRID,),
        in_specs=[
            dspec,
            pl.BlockSpec((_R, _D), lambda i: (i, 0)),
            pl.BlockSpec((_H, _D), lambda i: (0, 0)),
            pl.BlockSpec((1, _H), lambda i: (0, 0)),
            pl.BlockSpec((_H, _H), lambda i: (0, 0)),
        ],
        out_specs=[rspec, rspec],
        out_shape=[jax.ShapeDtypeStruct((_N, _H), f32)] * 2,
    )(degp3, node_feature, W0, b0.reshape(1, _H), Wc0)

    s0_p = _edge_pass(row2d, col2d, y0, zero2)

    h0r, y1 = pl.pallas_call(
        _mid_body,
        grid=(_GRID,),
        in_specs=[
            dspec, rspec, rspec, sspec,
            pl.BlockSpec((1, _H), lambda i: (0, 0)),
            pl.BlockSpec((_H, 2 * _H), lambda i: (0, 0)),
        ],
        out_specs=[rspec, rspec],
        out_shape=[jax.ShapeDtypeStruct((_N, _H), f32)] * 2,
    )(degp3, x, y0, s0_p, bc0.reshape(1, _H), Wc1)

    s1_p = _edge_pass(row2d, col2d, y1, zero2)

    out = pl.pallas_call(
        _fin_body,
        grid=(_GRID,),
        in_specs=[
            dspec, rspec, rspec, rspec, sspec,
            pl.BlockSpec((1, _H), lambda i: (0, 0)),
            pl.BlockSpec((_R, 1), lambda i: (i, 0)),
            pl.BlockSpec((_H, 3 * _H), lambda i: (0, 0)),
            pl.BlockSpec((1, _H), lambda i: (0, 0)),
            pl.BlockSpec((_OUT, _H), lambda i: (0, 0)),
            pl.BlockSpec((1, _OUT), lambda i: (0, 0)),
        ],
        out_specs=pl.BlockSpec((_G, _OUT), lambda i: (0, 0)),
        out_shape=jax.ShapeDtypeStruct((_G, _OUT), f32),
        scratch_shapes=[pltpu.VMEM((_G, 3 * _H), f32)],
    )(degp3, x, h0r, y1, s1_p, bc1.reshape(1, _H),
      batch.reshape(_N, 1), Wp1, bp1.reshape(1, _H), Wp2,
      bp2.reshape(1, _OUT))
    return out


# speedup vs baseline: 3.2461x; 2.8546x over previous
import functools

import jax
import jax.numpy as jnp
from jax import lax
from jax.experimental import pallas as pl
from jax.experimental.pallas import tpu as pltpu
from jax.experimental.pallas import tpu_sc as plsc

_N = 10000
_E = 320000
_D = 128
_H = 128
_OUT = 32
_G = 64

_NCORE = 2
_NSUB = 16
_NW = _NCORE * _NSUB
_NPAD = 10240
_RPS = _NPAD // _NSUB
_CHUNK = 128
_CPW = 80
_EPAD = _NW * _CPW * _CHUNK
_HCP = 40

_mesh = plsc.VectorSubcoreMesh(core_axis_name="c", subcore_axis_name="s")


@functools.partial(
    pl.kernel,
    out_type=jax.ShapeDtypeStruct((_NCORE, _NPAD), jnp.float32),
    mesh=_mesh,
    scratch_types=[
        pltpu.VMEM((_CPW, _CHUNK), jnp.int32),
        pltpu.VMEM((_CHUNK,), jnp.float32),
        pltpu.VMEM_SHARED((_NPAD,), jnp.float32),
    ],
)
def _deg_pass(col2d_hbm, zero1_hbm, out_hbm, cidx, ones_v, acc):
    c = lax.axis_index("c")
    s = lax.axis_index("s")
    w = s * _NCORE + c
    for i in range(_CHUNK // 16):
        ones_v[pl.ds(i * 16, 16)] = jnp.ones((16,), jnp.float32)
    pltpu.sync_copy(col2d_hbm.at[pl.ds(w * _CPW, _CPW)], cidx)
    pltpu.sync_copy(zero1_hbm, acc.at[pl.ds(s * _RPS, _RPS)])
    plsc.subcore_barrier()

    def body(k, carry):
        pltpu.sync_copy(ones_v, acc.at[cidx.at[k]], add=True)
        return carry

    lax.fori_loop(0, _CPW, body, 0)
    plsc.subcore_barrier()
    pltpu.sync_copy(acc.at[pl.ds(s * _RPS, _RPS)],
                    out_hbm.at[c, pl.ds(s * _RPS, _RPS)])


@functools.partial(
    pl.kernel,
    out_type=jax.ShapeDtypeStruct((_NCORE, _NPAD, _H), jnp.float32),
    mesh=_mesh,
    scratch_types=[
        pltpu.VMEM((_HCP, _CHUNK), jnp.int32),
        pltpu.VMEM((_HCP, _CHUNK), jnp.int32),
        pltpu.VMEM((_CHUNK, _H), jnp.float32),
        pltpu.VMEM((_CHUNK, _H), jnp.float32),
        pltpu.VMEM_SHARED((_NPAD, _H), jnp.float32),
        pltpu.SemaphoreType.DMA,
        pltpu.SemaphoreType.DMA,
    ],
)
def _edge_pass(row2d_hbm, col2d_hbm, y_hbm, zero2_hbm, out_hbm,
               ridx, cidx, rows_a, rows_b, acc, sem_a, sem_b):
    c = lax.axis_index("c")
    s = lax.axis_index("s")
    w = s * _NCORE + c
    pltpu.sync_copy(zero2_hbm, acc.at[pl.ds(s * _RPS, _RPS)])
    plsc.subcore_barrier()

    for h in range(_CPW // _HCP):
        base = w * _CPW + h * _HCP
        pltpu.sync_copy(row2d_hbm.at[pl.ds(base, _HCP)], ridx)
        pltpu.sync_copy(col2d_hbm.at[pl.ds(base, _HCP)], cidx)
        pltpu.async_copy(y_hbm.at[ridx.at[0]], rows_a, sem_a)

        def body(k2, carry):
            kk = k2 * 2
            pltpu.async_copy(y_hbm.at[ridx.at[kk + 1]], rows_b, sem_b)
            pltpu.make_async_copy(y_hbm.at[ridx.at[kk]], rows_a,
                                  sem_a).wait()
            pltpu.sync_copy(rows_a, acc.at[cidx.at[kk]], add=True)
            pltpu.async_copy(y_hbm.at[ridx.at[kk + 2]], rows_a, sem_a)
            pltpu.make_async_copy(y_hbm.at[ridx.at[kk + 1]], rows_b,
                                  sem_b).wait()
            pltpu.sync_copy(rows_b, acc.at[cidx.at[kk + 1]], add=True)
            return carry

        lax.fori_loop(0, _HCP // 2 - 1, body, 0)
        pltpu.async_copy(y_hbm.at[ridx.at[_HCP - 1]], rows_b, sem_b)
        pltpu.make_async_copy(y_hbm.at[ridx.at[_HCP - 2]], rows_a,
                              sem_a).wait()
        pltpu.sync_copy(rows_a, acc.at[cidx.at[_HCP - 2]], add=True)
        pltpu.make_async_copy(y_hbm.at[ridx.at[_HCP - 1]], rows_b,
                              sem_b).wait()
        pltpu.sync_copy(rows_b, acc.at[cidx.at[_HCP - 1]], add=True)

    plsc.subcore_barrier()
    pltpu.sync_copy(acc.at[pl.ds(s * _RPS, _RPS)],
                    out_hbm.at[c, pl.ds(s * _RPS, _RPS)])


_R = 1000
_GRID = _N // _R


def _pre_body(degp, nf, w0, b0, wc0, xo, y0o):
    d = degp[...]
    dinv = lax.rsqrt(d[0] + d[1] + 1.0)
    x = lax.dot_general(nf[...], w0[...], (((1,), (1,)), ((), ())),
                        preferred_element_type=jnp.float32) + b0[...]
    xo[...] = x
    h0 = lax.dot_general(x, wc0[...], (((1,), (1,)), ((), ())),
                         preferred_element_type=jnp.float32)
    y0o[...] = dinv * h0


def _mid_body(degp, x, y0, s0p, bc0, wc1, h0ro, y1o):
    d = degp[...]
    dinv = lax.rsqrt(d[0] + d[1] + 1.0)
    sp = s0p[...]
    t = dinv * (sp[0] + sp[1] + y0[...]) + bc0[...]
    h0r = jnp.maximum(t, 0.0)
    h0ro[...] = h0r
    emb = jnp.concatenate([x[...], h0r], axis=1)
    h1 = lax.dot_general(emb, wc1[...], (((1,), (1,)), ((), ())),
                         preferred_element_type=jnp.float32)
    y1o[...] = dinv * h1


def _fin_body(degp, x, h0r, y1, s1p, bc1, bt, wp1, bp1, wp2, bp2,
              out, pooled):
    i = pl.program_id(0)
    d = degp[...]
    dinv = lax.rsqrt(d[0] + d[1] + 1.0)
    sp = s1p[...]
    t = dinv * (sp[0] + sp[1] + y1[...]) + bc1[...]
    h1r = jnp.maximum(t, 0.0)
    emb = jnp.concatenate([x[...], h0r[...], h1r], axis=1)
    seg = lax.broadcasted_iota(jnp.int32, (_R, _G), 1)
    onehot = jnp.where(bt[...] == seg, 1.0, 0.0).astype(jnp.float32)
    part = lax.dot_general(onehot, emb, (((0,), (0,)), ((), ())),
                           preferred_element_type=jnp.float32)

    @pl.when(i == 0)
    def _():
        pooled[...] = part

    @pl.when(i > 0)
    def _():
        pooled[...] = pooled[...] + part

    @pl.when(i == _GRID - 1)
    def _():
        p = pooled[...]
        h = lax.dot_general(p, wp1[...], (((1,), (1,)), ((), ())),
                            preferred_element_type=jnp.float32) + bp1[...]
        h = jnp.where(h > 0, h, 0.1 * h)
        o = lax.dot_general(h, wp2[...], (((1,), (1,)), ((), ())),
                            preferred_element_type=jnp.float32) + bp2[...]
        m = jnp.max(o, axis=1, keepdims=True)
        lse = jnp.log(jnp.sum(jnp.exp(o - m), axis=1, keepdims=True)) + m
        out[...] = o - lse


def kernel(node_feature, edge_index, batch, W0, b0, Wc0, bc0, Wc1, bc1,
           Wp1, bp1, Wp2, bp2):
    f32 = jnp.float32
    i32 = jnp.int32
    pad = _EPAD - _E
    prow = jnp.arange(pad, dtype=i32) * 125 % _N
    row2d = jnp.concatenate([edge_index[0], prow]).reshape(-1, _CHUNK)
    dump = _N + (jnp.arange(pad, dtype=i32) % (_NPAD - _N))
    col2d = jnp.concatenate([edge_index[1], dump]).reshape(-1, _CHUNK)
    zero1 = jnp.zeros((_RPS,), f32)
    zero2 = jnp.zeros((_RPS, _H), f32)

    deg_p = _deg_pass(col2d, zero1)
    degp3 = deg_p.reshape(_NCORE, _NPAD, 1)

    dspec = pl.BlockSpec((_NCORE, _R, 1), lambda i: (0, i, 0))
    rspec = pl.BlockSpec((_R, _H), lambda i: (i, 0))
    sspec = pl.BlockSpec((_NCORE, _R, _H), lambda i: (0, i, 0))

    x, y0 = pl.pallas_call(
        _pre_body,
        grid=(_GRID,),
        in_specs=[
            dspec,
            pl.BlockSpec((_R, _D), lambda i: (i, 0)),
            pl.BlockSpec((_H, _D), lambda i: (0, 0)),
            pl.BlockSpec((1, _H), lambda i: (0, 0)),
            pl.BlockSpec((_H, _H), lambda i: (0, 0)),
        ],
        out_specs=[rspec, rspec],
        out_shape=[jax.ShapeDtypeStruct((_N, _H), f32)] * 2,
    )(degp3, node_feature, W0, b0.reshape(1, _H), Wc0)

    s0_p = _edge_pass(row2d, col2d, y0, zero2)

    h0r, y1 = pl.pallas_call(
        _mid_body,
        grid=(_GRID,),
        in_specs=[
            dspec, rspec, rspec, sspec,
            pl.BlockSpec((1, _H), lambda i: (0, 0)),
            pl.BlockSpec((_H, 2 * _H), lambda i: (0, 0)),
        ],
        out_specs=[rspec, rspec],
        out_shape=[jax.ShapeDtypeStruct((_N, _H), f32)] * 2,
    )(degp3, x, y0, s0_p, bc0.reshape(1, _H), Wc1)

    s1_p = _edge_pass(row2d, col2d, y1, zero2)

    out = pl.pallas_call(
        _fin_body,
        grid=(_GRID,),
        in_specs=[
            dspec, rspec, rspec, rspec, sspec,
            pl.BlockSpec((1, _H), lambda i: (0, 0)),
            pl.BlockSpec((_R, 1), lambda i: (i, 0)),
            pl.BlockSpec((_H, 3 * _H), lambda i: (0, 0)),
            pl.BlockSpec((1, _H), lambda i: (0, 0)),
            pl.BlockSpec((_OUT, _H), lambda i: (0, 0)),
            pl.BlockSpec((1, _OUT), lambda i: (0, 0)),
        ],
        out_specs=pl.BlockSpec((_G, _OUT), lambda i: (0, 0)),
        out_shape=jax.ShapeDtypeStruct((_G, _OUT), f32),
        scratch_shapes=[pltpu.VMEM((_G, 3 * _H), f32)],
    )(degp3, x, h0r, y1, s1_p, bc1.reshape(1, _H),
      batch.reshape(_N, 1), Wp1, bp1.reshape(1, _H), Wp2,
      bp2.reshape(1, _OUT))
    return out


# speedup vs baseline: 3.3845x; 1.0426x over previous
import functools

import jax
import jax.numpy as jnp
from jax import lax
from jax.experimental import pallas as pl
from jax.experimental.pallas import tpu as pltpu
from jax.experimental.pallas import tpu_sc as plsc

_N = 10000
_E = 320000
_D = 128
_H = 128
_OUT = 32
_G = 64

_NCORE = 2
_NSUB = 16
_NW = _NCORE * _NSUB
_NPAD = 10240
_RPS = _NPAD // _NSUB
_CHUNK = 128
_NCH = _E // _CHUNK
_CPW = 80
_HCP = 40
_LASTW = _NCH // _CPW
_LASTN = _NCH - _LASTW * _CPW

_mesh = plsc.VectorSubcoreMesh(core_axis_name="c", subcore_axis_name="s")


@functools.partial(
    pl.kernel,
    out_type=jax.ShapeDtypeStruct((_NCORE, _NPAD), jnp.float32),
    mesh=_mesh,
    scratch_types=[
        pltpu.VMEM((_CPW, _CHUNK), jnp.int32),
        pltpu.VMEM((_CHUNK,), jnp.float32),
        pltpu.VMEM_SHARED((_NPAD,), jnp.float32),
    ],
)
def _deg_pass(edge3d_hbm, zero1_hbm, out_hbm, cidx, ones_v, acc):
    c = lax.axis_index("c")
    s = lax.axis_index("s")
    w = s * _NCORE + c
    for i in range(_CHUNK // 16):
        ones_v[pl.ds(i * 16, 16)] = jnp.ones((16,), jnp.float32)
    pltpu.sync_copy(zero1_hbm, acc.at[pl.ds(s * _RPS, _RPS)])

    @pl.when(w < _LASTW)
    def _():
        pltpu.sync_copy(edge3d_hbm.at[1, pl.ds(w * _CPW, _CPW)], cidx)

    @pl.when(w == _LASTW)
    def _():
        pltpu.sync_copy(edge3d_hbm.at[1, pl.ds(_LASTW * _CPW, _LASTN)],
                        cidx.at[pl.ds(0, _LASTN)])

    plsc.subcore_barrier()

    def body(k, carry):
        pltpu.sync_copy(ones_v, acc.at[cidx.at[k]], add=True)
        return carry

    nch = jnp.where(w < _LASTW, _CPW, _LASTN)
    lax.fori_loop(0, nch, body, 0)
    plsc.subcore_barrier()
    pltpu.sync_copy(acc.at[pl.ds(s * _RPS, _RPS)],
                    out_hbm.at[c, pl.ds(s * _RPS, _RPS)])


@functools.partial(
    pl.kernel,
    out_type=jax.ShapeDtypeStruct((_NCORE, _NPAD, _H), jnp.float32),
    mesh=_mesh,
    scratch_types=[
        pltpu.VMEM((_HCP, _CHUNK), jnp.int32),
        pltpu.VMEM((_HCP, _CHUNK), jnp.int32),
        pltpu.VMEM((_CHUNK, _H), jnp.float32),
        pltpu.VMEM((_CHUNK, _H), jnp.float32),
        pltpu.VMEM_SHARED((_NPAD, _H), jnp.float32),
        pltpu.SemaphoreType.DMA,
        pltpu.SemaphoreType.DMA,
    ],
)
def _edge_pass(edge3d_hbm, y_hbm, zero2_hbm, out_hbm,
               ridx, cidx, rows_a, rows_b, acc, sem_a, sem_b):
    c = lax.axis_index("c")
    s = lax.axis_index("s")
    w = s * _NCORE + c
    pltpu.sync_copy(zero2_hbm, acc.at[pl.ds(s * _RPS, _RPS)])
    plsc.subcore_barrier()

    def pipeline(n):
        pltpu.async_copy(y_hbm.at[ridx.at[0]], rows_a, sem_a)

        def body(k2, carry):
            kk = k2 * 2
            pltpu.async_copy(y_hbm.at[ridx.at[kk + 1]], rows_b, sem_b)
            pltpu.make_async_copy(y_hbm.at[ridx.at[kk]], rows_a,
                                  sem_a).wait()
            pltpu.sync_copy(rows_a, acc.at[cidx.at[kk]], add=True)
            pltpu.async_copy(y_hbm.at[ridx.at[kk + 2]], rows_a, sem_a)
            pltpu.make_async_copy(y_hbm.at[ridx.at[kk + 1]], rows_b,
                                  sem_b).wait()
            pltpu.sync_copy(rows_b, acc.at[cidx.at[kk + 1]], add=True)
            return carry

        lax.fori_loop(0, n // 2 - 1, body, 0)
        pltpu.async_copy(y_hbm.at[ridx.at[n - 1]], rows_b, sem_b)
        pltpu.make_async_copy(y_hbm.at[ridx.at[n - 2]], rows_a,
                              sem_a).wait()
        pltpu.sync_copy(rows_a, acc.at[cidx.at[n - 2]], add=True)
        pltpu.make_async_copy(y_hbm.at[ridx.at[n - 1]], rows_b,
                              sem_b).wait()
        pltpu.sync_copy(rows_b, acc.at[cidx.at[n - 1]], add=True)

    for h in range(_CPW // _HCP):
        @pl.when(w < _LASTW)
        def _():
            base = w * _CPW + h * _HCP
            pltpu.sync_copy(edge3d_hbm.at[0, pl.ds(base, _HCP)], ridx)
            pltpu.sync_copy(edge3d_hbm.at[1, pl.ds(base, _HCP)], cidx)
            pipeline(_HCP)

        if h == 0:
            @pl.when(w == _LASTW)
            def _():
                base = _LASTW * _CPW
                pltpu.sync_copy(edge3d_hbm.at[0, pl.ds(base, _LASTN)],
                                ridx.at[pl.ds(0, _LASTN)])
                pltpu.sync_copy(edge3d_hbm.at[1, pl.ds(base, _LASTN)],
                                cidx.at[pl.ds(0, _LASTN)])
                pipeline(_LASTN)

    plsc.subcore_barrier()
    pltpu.sync_copy(acc.at[pl.ds(s * _RPS, _RPS)],
                    out_hbm.at[c, pl.ds(s * _RPS, _RPS)])


_R = 1000
_GRID = _N // _R


def _pre_a_body(nf, w0, b0, wc0, xo, h0o):
    x = lax.dot_general(nf[...], w0[...], (((1,), (1,)), ((), ())),
                        preferred_element_type=jnp.float32) + b0[...]
    xo[...] = x
    h0o[...] = lax.dot_general(x, wc0[...], (((1,), (1,)), ((), ())),
                               preferred_element_type=jnp.float32)


def _pre_b_body(degp, h0, y0o):
    d = degp[...]
    dinv = lax.rsqrt(d[0] + d[1] + 1.0)
    y0o[...] = dinv * h0[...]


def _mid_body(degp, x, y0, s0p, bc0, wc1, h0ro, y1o):
    d = degp[...]
    dinv = lax.rsqrt(d[0] + d[1] + 1.0)
    sp = s0p[...]
    t = dinv * (sp[0] + sp[1] + y0[...]) + bc0[...]
    h0r = jnp.maximum(t, 0.0)
    h0ro[...] = h0r
    emb = jnp.concatenate([x[...], h0r], axis=1)
    h1 = lax.dot_general(emb, wc1[...], (((1,), (1,)), ((), ())),
                         preferred_element_type=jnp.float32)
    y1o[...] = dinv * h1


def _fin_a_body(x, h0r, bt, pao, pacc):
    i = pl.program_id(0)
    emb = jnp.concatenate([x[...], h0r[...]], axis=1)
    seg = lax.broadcasted_iota(jnp.int32, (_R, _G), 1)
    onehot = jnp.where(bt[...] == seg, 1.0, 0.0).astype(jnp.float32)
    part = lax.dot_general(onehot, emb, (((0,), (0,)), ((), ())),
                           preferred_element_type=jnp.float32)

    @pl.when(i == 0)
    def _():
        pacc[...] = part

    @pl.when(i > 0)
    def _():
        pacc[...] = pacc[...] + part

    @pl.when(i == _GRID - 1)
    def _():
        pao[...] = pacc[...]


def _fin_b_body(degp, y1, s1p, bc1, bt, pa, wp1, bp1, wp2, bp2,
                out, pacc):
    i = pl.program_id(0)
    d = degp[...]
    dinv = lax.rsqrt(d[0] + d[1] + 1.0)
    sp = s1p[...]
    t = dinv * (sp[0] + sp[1] + y1[...]) + bc1[...]
    h1r = jnp.maximum(t, 0.0)
    seg = lax.broadcasted_iota(jnp.int32, (_R, _G), 1)
    onehot = jnp.where(bt[...] == seg, 1.0, 0.0).astype(jnp.float32)
    part = lax.dot_general(onehot, h1r, (((0,), (0,)), ((), ())),
                           preferred_element_type=jnp.float32)

    @pl.when(i == 0)
    def _():
        pacc[...] = part

    @pl.when(i > 0)
    def _():
        pacc[...] = pacc[...] + part

    @pl.when(i == _GRID - 1)
    def _():
        p = jnp.concatenate([pa[...], pacc[...]], axis=1)
        h = lax.dot_general(p, wp1[...], (((1,), (1,)), ((), ())),
                            preferred_element_type=jnp.float32) + bp1[...]
        h = jnp.where(h > 0, h, 0.1 * h)
        o = lax.dot_general(h, wp2[...], (((1,), (1,)), ((), ())),
                            preferred_element_type=jnp.float32) + bp2[...]
        m = jnp.max(o, axis=1, keepdims=True)
        lse = jnp.log(jnp.sum(jnp.exp(o - m), axis=1, keepdims=True)) + m
        out[...] = o - lse


def kernel(node_feature, edge_index, batch, W0, b0, Wc0, bc0, Wc1, bc1,
           Wp1, bp1, Wp2, bp2):
    f32 = jnp.float32
    edge3d = edge_index.reshape(2, _NCH, _CHUNK)
    zero1 = jnp.zeros((_RPS,), f32)
    zero2 = jnp.zeros((_RPS, _H), f32)
    bt2 = batch.reshape(_N, 1)

    dspec = pl.BlockSpec((_NCORE, _R, 1), lambda i: (0, i, 0))
    rspec = pl.BlockSpec((_R, _H), lambda i: (i, 0))
    sspec = pl.BlockSpec((_NCORE, _R, _H), lambda i: (0, i, 0))
    btspec = pl.BlockSpec((_R, 1), lambda i: (i, 0))

    deg_p = _deg_pass(edge3d, zero1)
    degp3 = deg_p.reshape(_NCORE, _NPAD, 1)

    x, h0 = pl.pallas_call(
        _pre_a_body,
        grid=(_GRID,),
        in_specs=[
            pl.BlockSpec((_R, _D), lambda i: (i, 0)),
            pl.BlockSpec((_H, _D), lambda i: (0, 0)),
            pl.BlockSpec((1, _H), lambda i: (0, 0)),
            pl.BlockSpec((_H, _H), lambda i: (0, 0)),
        ],
        out_specs=[rspec, rspec],
        out_shape=[jax.ShapeDtypeStruct((_N, _H), f32)] * 2,
    )(node_feature, W0, b0.reshape(1, _H), Wc0)

    y0 = pl.pallas_call(
        _pre_b_body,
        grid=(_GRID,),
        in_specs=[dspec, rspec],
        out_specs=rspec,
        out_shape=jax.ShapeDtypeStruct((_N, _H), f32),
    )(degp3, h0)

    s0_p = _edge_pass(edge3d, y0, zero2)

    h0r, y1 = pl.pallas_call(
        _mid_body,
        grid=(_GRID,),
        in_specs=[
            dspec, rspec, rspec, sspec,
            pl.BlockSpec((1, _H), lambda i: (0, 0)),
            pl.BlockSpec((_H, 2 * _H), lambda i: (0, 0)),
        ],
        out_specs=[rspec, rspec],
        out_shape=[jax.ShapeDtypeStruct((_N, _H), f32)] * 2,
    )(degp3, x, y0, s0_p, bc0.reshape(1, _H), Wc1)

    s1_p = _edge_pass(edge3d, y1, zero2)

    pooled_a = pl.pallas_call(
        _fin_a_body,
        grid=(_GRID,),
        in_specs=[rspec, rspec, btspec],
        out_specs=pl.BlockSpec((_G, 2 * _H), lambda i: (0, 0)),
        out_shape=jax.ShapeDtypeStruct((_G, 2 * _H), f32),
        scratch_shapes=[pltpu.VMEM((_G, 2 * _H), f32)],
    )(x, h0r, bt2)

    out = pl.pallas_call(
        _fin_b_body,
        grid=(_GRID,),
        in_specs=[
            dspec, rspec, sspec,
            pl.BlockSpec((1, _H), lambda i: (0, 0)),
            btspec,
            pl.BlockSpec((_G, 2 * _H), lambda i: (0, 0)),
            pl.BlockSpec((_H, 3 * _H), lambda i: (0, 0)),
            pl.BlockSpec((1, _H), lambda i: (0, 0)),
            pl.BlockSpec((_OUT, _H), lambda i: (0, 0)),
            pl.BlockSpec((1, _OUT), lambda i: (0, 0)),
        ],
        out_specs=pl.BlockSpec((_G, _OUT), lambda i: (0, 0)),
        out_shape=jax.ShapeDtypeStruct((_G, _OUT), f32),
        scratch_shapes=[pltpu.VMEM((_G, _H), f32)],
    )(degp3, y1, s1_p, bc1.reshape(1, _H), bt2, pooled_a,
      Wp1, bp1.reshape(1, _H), Wp2, bp2.reshape(1, _OUT))
    return out
